# baseline (speedup 1.0000x reference)
import jax
import jax.numpy as jnp
from jax import lax
from jax.experimental import pallas as pl
from jax.experimental.pallas import tpu as pltpu

_K = 64
_N = 32768
_BLOCK_R = 64
_CELL_M = 4
_LANE_M = 8


def _to_key(v):
    iv = lax.bitcast_convert_type(v, jnp.int32)
    return jnp.where(iv < 0, iv ^ jnp.int32(0x7FFFFFFF), iv)


def _count_ge(key, m):
    return jnp.sum((key >= m).astype(jnp.int32), axis=1, keepdims=True)


def _search_kth(key, k):
    n_pos = _count_ge(key, jnp.zeros((key.shape[0], 1), jnp.int32))
    pos = n_pos >= k
    lo = jnp.where(pos, jnp.int32(0), jnp.int32(-(2**31)))
    hi = jnp.where(pos, jnp.int32(2**31 - 1), jnp.int32(-1))

    def body(_, carry):
        lo, hi = carry
        mid = lo + lax.shift_right_logical(hi - lo, 1) + 1
        ge = _count_ge(key, mid) >= k
        return jnp.where(ge, mid, lo), jnp.where(ge, hi, mid - 1)

    lo, hi = lax.fori_loop(0, 31, body, (lo, hi), unroll=False)
    return lo


def _wta_block(x_ref, o_ref):
    x = x_ref[...]
    R = x.shape[0]
    neg = jnp.float32(-jnp.inf)

    xv = x.reshape(R, 32, 8, 128)
    ta = [jnp.full((R, 1, 8, 128), neg) for _ in range(_CELL_M)]
    for s in range(32):
        v = xv[:, s:s + 1]
        for j in range(_CELL_M):
            nt = jnp.maximum(ta[j], v)
            if j < _CELL_M - 1:
                v = jnp.minimum(ta[j], v)
            ta[j] = nt
    cand0 = jnp.concatenate(ta, axis=1).reshape(R, _CELL_M * 8, 128)

    c1 = []
    for _ in range(_LANE_M):
        cm = jnp.max(cand0, axis=1, keepdims=True)
        c1.append(cm)
        cand0 = jnp.where(cand0 == cm, neg, cand0)
    cand = jnp.concatenate(c1, axis=1).reshape(R, _LANE_M * 128)

    ckey = _to_key(cand)
    ck3 = ckey.reshape(R, _LANE_M, 128)
    lo0 = jnp.min(ck3[:, 0, :], axis=1, keepdims=True)
    hi0 = jnp.max(ck3[:, 0, :], axis=1, keepdims=True)

    def scond(carry):
        lo, hi = carry
        return jnp.any(lo < hi)

    def sbody(carry):
        lo, hi = carry
        mid = lo + lax.shift_right_logical(hi - lo, 1) + 1
        ge_ = jnp.sum((ckey >= mid).astype(jnp.int32), axis=1,
                      keepdims=True) >= _K
        return jnp.where(ge_, mid, lo), jnp.where(ge_, hi, mid - 1)

    t_key, _ = lax.while_loop(scond, sbody, (lo0, hi0))
    t_f = lax.bitcast_convert_type(
        jnp.where(t_key < 0, t_key ^ jnp.int32(0x7FFFFFFF), t_key),
        jnp.float32)

    ge = x >= t_f
    n_ge = jnp.sum(ge.astype(jnp.int32), axis=1, keepdims=True)
    fast = jnp.all(n_ge == _K)

    @pl.when(fast)
    def _fast_path():
        o_ref[...] = jnp.where(ge, x, 0.0)

    @pl.when(jnp.logical_not(fast))
    def _slow_path():
        key = _to_key(x)
        kstar = _search_kth(key, _K)
        gt = key > kstar
        eq = key == kstar
        needed = _K - jnp.sum(gt.astype(jnp.int32), axis=1, keepdims=True)
        idx = lax.broadcasted_iota(jnp.int32, key.shape, 1)
        ilo = jnp.zeros((R, 1), jnp.int32)
        ihi = jnp.full((R, 1), _N - 1, jnp.int32)

        def ibody(_, carry):
            ilo, ihi = carry
            mid = ilo + lax.shift_right_logical(ihi - ilo, 1)
            cnt = jnp.sum((eq & (idx <= mid)).astype(jnp.int32), axis=1,
                          keepdims=True)
            take = cnt >= needed
            return jnp.where(take, ilo, mid + 1), jnp.where(take, mid, ihi)

        ilo, _ = lax.fori_loop(0, 15, ibody, (ilo, ihi), unroll=False)
        sel = gt | (eq & (idx <= ilo))
        o_ref[...] = jnp.where(sel, x, 0.0)


def kernel(x):
    B, N = x.shape
    grid = (B // _BLOCK_R,)
    return pl.pallas_call(
        _wta_block,
        grid=grid,
        in_specs=[pl.BlockSpec((_BLOCK_R, N), lambda i: (i, 0))],
        out_specs=pl.BlockSpec((_BLOCK_R, N), lambda i: (i, 0)),
        out_shape=jax.ShapeDtypeStruct((B, N), x.dtype),
    )(x)


# speedup vs baseline: 1.0848x; 1.0848x over previous
import jax
import jax.numpy as jnp
from jax import lax
from jax.experimental import pallas as pl
from jax.experimental.pallas import tpu as pltpu

_K = 64
_N = 32768
_BLOCK_R = 64
_CELL_M = 4
_LANE_M = 8


def _to_key(v):
    iv = lax.bitcast_convert_type(v, jnp.int32)
    return jnp.where(iv < 0, iv ^ jnp.int32(0x7FFFFFFF), iv)


def _count_ge(key, m):
    return jnp.sum((key >= m).astype(jnp.int32), axis=1, keepdims=True)


def _search_kth(key, k):
    n_pos = _count_ge(key, jnp.zeros((key.shape[0], 1), jnp.int32))
    pos = n_pos >= k
    lo = jnp.where(pos, jnp.int32(0), jnp.int32(-(2**31)))
    hi = jnp.where(pos, jnp.int32(2**31 - 1), jnp.int32(-1))

    def body(_, carry):
        lo, hi = carry
        q = lax.shift_right_logical(hi - lo, 2)
        m1 = lo + q + 1
        m2 = lo + 2 * q + 1
        m3 = lo + 3 * q + 1
        c1 = _count_ge(key, m1) >= k
        c2 = _count_ge(key, m2) >= k
        c3 = _count_ge(key, m3) >= k
        lo = jnp.where(c3, m3, jnp.where(c2, m2, jnp.where(c1, m1, lo)))
        hi = jnp.where(c3, hi, jnp.where(c2, m3 - 1,
                                         jnp.where(c1, m2 - 1, m1 - 1)))
        return lo, hi

    lo, hi = lax.fori_loop(0, 16, body, (lo, hi), unroll=False)
    return lo


def _wta_block(x_ref, o_ref):
    x = x_ref[...]
    R = x.shape[0]
    neg = jnp.float32(-jnp.inf)

    xv = x.reshape(R, 32, 8, 128)
    ta = [jnp.full((R, 1, 8, 128), neg) for _ in range(_CELL_M)]
    for s in range(32):
        v = xv[:, s:s + 1]
        for j in range(_CELL_M):
            nt = jnp.maximum(ta[j], v)
            if j < _CELL_M - 1:
                v = jnp.minimum(ta[j], v)
            ta[j] = nt
    cand0 = jnp.concatenate(ta, axis=1).reshape(R, _CELL_M * 8, 128)

    c1 = []
    for _ in range(_LANE_M):
        cm = jnp.max(cand0, axis=1, keepdims=True)
        c1.append(cm)
        cand0 = jnp.where(cand0 == cm, neg, cand0)
    cand = jnp.concatenate(c1, axis=1).reshape(R, _LANE_M * 128)

    t_key = _search_kth(_to_key(cand), _K)
    t_f = lax.bitcast_convert_type(
        jnp.where(t_key < 0, t_key ^ jnp.int32(0x7FFFFFFF), t_key),
        jnp.float32)

    ge = x >= t_f
    n_ge = jnp.sum(ge.astype(jnp.int32), axis=1, keepdims=True)
    fast = jnp.all(n_ge == _K)

    @pl.when(fast)
    def _fast_path():
        o_ref[...] = jnp.where(ge, x, 0.0)

    @pl.when(jnp.logical_not(fast))
    def _slow_path():
        key = _to_key(x)
        kstar = _search_kth(key, _K)
        gt = key > kstar
        eq = key == kstar
        needed = _K - jnp.sum(gt.astype(jnp.int32), axis=1, keepdims=True)
        idx = lax.broadcasted_iota(jnp.int32, key.shape, 1)
        ilo = jnp.zeros((R, 1), jnp.int32)
        ihi = jnp.full((R, 1), _N - 1, jnp.int32)

        def ibody(_, carry):
            ilo, ihi = carry
            mid = ilo + lax.shift_right_logical(ihi - ilo, 1)
            cnt = jnp.sum((eq & (idx <= mid)).astype(jnp.int32), axis=1,
                          keepdims=True)
            take = cnt >= needed
            return jnp.where(take, ilo, mid + 1), jnp.where(take, mid, ihi)

        ilo, _ = lax.fori_loop(0, 15, ibody, (ilo, ihi), unroll=False)
        sel = gt | (eq & (idx <= ilo))
        o_ref[...] = jnp.where(sel, x, 0.0)


def kernel(x):
    B, N = x.shape
    grid = (B // _BLOCK_R,)
    return pl.pallas_call(
        _wta_block,
        grid=grid,
        in_specs=[pl.BlockSpec((_BLOCK_R, N), lambda i: (i, 0))],
        out_specs=pl.BlockSpec((_BLOCK_R, N), lambda i: (i, 0)),
        out_shape=jax.ShapeDtypeStruct((B, N), x.dtype),
    )(x)
